# baseline (speedup 1.0000x reference)
import numpy as np

import jax
import jax.numpy as jnp
from jax import lax
from jax.experimental import pallas as pl
from jax.experimental.pallas import tpu as pltpu
from jax.experimental.pallas import tpu_sc as plsc

N_NODES = 10000
D = 128
DW = D // 2
E = 320000
N_HOPS_K = 3

NC = 2
NS = 16
NW = NC * NS
EPT = E // NW
C = 64
NCHUNK = 160
EPAD = NCHUNK * C
BLK = 16
NBLK = NCHUNK // BLK

RPT = 624
ZB = 16
NZ = RPT // ZB
ZR = 104
TAIL = N_NODES - NS * RPT
TAIL_OFF = NS * RPT

RPC = 312
CTAIL_OFF = NW * RPC

_MESH = plsc.VectorSubcoreMesh(
    core_axis_name="c", subcore_axis_name="s", num_cores=NC, num_subcores=NS
)

_PERM32 = np.concatenate([np.arange(0, 32, 2), np.arange(1, 32, 2)])
_PERM = np.concatenate([32 * b + _PERM32 for b in range(D // 32)])
_P1 = _PERM
_P2 = _PERM[_PERM]
_P3 = _PERM[_PERM[_PERM]]
_INV1 = np.argsort(_P1)
_INV2 = np.argsort(_P2)
_INV3 = np.argsort(_P3)


def _hop_body(agg, pk, trf, part,
              pkq0, pkq1, trq0, trq1,
              rb0, rb1, rb2, rb3, cb0, cb1, cb2, cb3, tf0, tf1, tf2, tf3,
              gb0, gb1, sb0, sb1, zbuf, acc,
              qs0, qs1, gs0, gs1, ss0, ss1):
    cid = lax.axis_index("c")
    sid = lax.axis_index("s")
    wid = cid * NS + sid

    pkq = (pkq0, pkq1)
    trq = (trq0, trq1)
    rb = (rb0, rb1, rb2, rb3)
    cb = (cb0, cb1, cb2, cb3)
    tf = (tf0, tf1, tf2, tf3)
    gb = (gb0, gb1)
    sb = (sb0, sb1)
    qsem = (qs0, qs1)
    gsem = (gs0, gs1)
    ssem = (ss0, ss1)

    def zb_fill(j, carry):
        for k in range(D // 16):
            zbuf[j, pl.ds(k * 16, 16)] = jnp.zeros((16,), jnp.float32)
        return carry
    lax.fori_loop(0, ZB, zb_fill, 0)

    def za(k, carry):
        pltpu.sync_copy(zbuf, acc.at[pl.ds(sid * RPT + k * ZB, ZB)])
        return carry
    lax.fori_loop(0, NZ, za, 0)

    @pl.when(sid == NS - 1)
    def _():
        pltpu.sync_copy(zbuf.at[pl.ds(0, TAIL)], acc.at[pl.ds(TAIL_OFF, TAIL)])
    plsc.subcore_barrier()

    pltpu.sync_copy(pk.at[wid, 0], pkq0)
    pltpu.sync_copy(trf.at[wid, 0], trq0)
    pltpu.async_copy(pk.at[wid, 1], pkq1, qs1)
    pltpu.async_copy(trf.at[wid, 1], trq1, qs1)

    def unpack(cc, s):
        blk = cc // BLK
        lc = cc - blk * BLK
        qsel = lax.rem(blk, 2)
        for q in range(2):
            @pl.when(qsel == q)
            def _():
                for w in range(C // 16):
                    v = pkq[q][lc, pl.ds(w * 16, 16)]
                    rb[s][pl.ds(w * 16, 16)] = v & 0xFFFF
                    cb[s][pl.ds(w * 16, 16)] = lax.shift_right_logical(v, 16)
                    tf[s][pl.ds(w * 16, 16)] = trq[q][lc, pl.ds(w * 16, 16)]

    def gather_start(s, p):
        pltpu.async_copy(agg.at[rb[s]], gb[p], gsem[p])

    def gather_wait(s, p):
        pltpu.make_async_copy(agg.at[rb[s]], gb[p], gsem[p]).wait()

    def scatter_start(s, p):
        pltpu.sync_copy(sb[p], acc.at[cb[s]], add=True)

    def scatter_wait(s, p):
        pass

    def scale(s, p):
        src = gb[p]
        dst = sb[p]
        tr_ref = tf[s]
        hi_mask = jnp.int32(-65536)

        def grp(j16, carry):
            t16 = tr_ref[pl.ds(j16 * 16, 16)]
            for jj in range(16):
                tbc = lax.broadcast(t16[jj], (16,))
                j = j16 * 16 + jj
                for w in range(DW // 16):
                    v = src[j, pl.ds(w * 16, 16)]
                    fa = plsc.bitcast(lax.shift_left(v, 16), jnp.float32)
                    fb = plsc.bitcast(v & hi_mask, jnp.float32)
                    dst[j, pl.ds(32 * w, 16)] = fa * tbc
                    dst[j, pl.ds(32 * w + 16, 16)] = fb * tbc
            return carry
        lax.fori_loop(0, C // 16, grp, 0)

    def blkmgmt(c):
        blk = c // BLK
        lc = c - blk * BLK
        nq = lax.rem(blk + 1, 2)

        @pl.when(jnp.logical_and(lc == 2,
                                 jnp.logical_and(c > BLK, blk < NBLK - 1)))
        def _():
            for q in range(2):
                @pl.when(nq == q)
                def _():
                    pltpu.async_copy(pk.at[wid, blk + 1], pkq[q], qsem[q])
                    pltpu.async_copy(trf.at[wid, blk + 1], trq[q], qsem[q])

        @pl.when(jnp.logical_and(lc == 14, blk < NBLK - 1))
        def _():
            for q in range(2):
                @pl.when(nq == q)
                def _():
                    pltpu.make_async_copy(pk.at[wid, 0], pkq[q], qsem[q]).wait()
                    pltpu.make_async_copy(trf.at[wid, 0], trq[q], qsem[q]).wait()

    def quadbody(i4, carry):
        c = 4 * i4

        @pl.when(i4 == 0)
        def _():
            unpack(0, 0)
            gather_start(0, 0)

        for k in range(4):
            cc = c + k
            p = k % 2
            blkmgmt(cc)
            if k < 3:
                unpack(cc + 1, k + 1)
                gather_start(k + 1, 1 - p)
            else:
                @pl.when(cc + 1 < NCHUNK)
                def _():
                    unpack(cc + 1, 0)
                    gather_start(0, 0)
            gather_wait(k, p)
            if k < 2:
                @pl.when(i4 > 0)
                def _():
                    scatter_wait((k + 2) % 4, p)
            else:
                scatter_wait((k + 2) % 4, p)
            scale(k, p)
            scatter_start(k, p)
        return carry
    lax.fori_loop(0, NCHUNK // 4, quadbody, 0)
    scatter_wait(2, 0)
    scatter_wait(3, 1)

    plsc.subcore_barrier()
    pltpu.sync_copy(acc.at[pl.ds(sid * RPT, RPT)],
                    part.at[cid, pl.ds(sid * RPT, RPT)])

    @pl.when(sid == NS - 1)
    def _():
        pltpu.sync_copy(acc.at[pl.ds(TAIL_OFF, TAIL)],
                        part.at[cid, pl.ds(TAIL_OFF, TAIL)])


def _combine_body(part, out, b0, b1):
    cid = lax.axis_index("c")
    sid = lax.axis_index("s")
    wid = cid * NS + sid

    def _sum_rows(nrows, off):
        pltpu.sync_copy(part.at[0, pl.ds(off, nrows)], b0.at[pl.ds(0, nrows)])
        pltpu.sync_copy(part.at[1, pl.ds(off, nrows)], b1.at[pl.ds(0, nrows)])

        def addrow(j, c2):
            for kk in range(D // 16):
                b0[j, pl.ds(kk * 16, 16)] = (
                    b0[j, pl.ds(kk * 16, 16)] + b1[j, pl.ds(kk * 16, 16)])
            return c2
        lax.fori_loop(0, nrows, addrow, 0)
        pltpu.sync_copy(b0.at[pl.ds(0, nrows)], out.at[pl.ds(off, nrows)])

    def ck(k, carry):
        _sum_rows(ZR, wid * RPC + k * ZR)
        return carry
    lax.fori_loop(0, RPC // ZR, ck, 0)

    @pl.when(wid == NW - 1)
    def _():
        _sum_rows(TAIL, CTAIL_OFF)


_SC_PARAMS = pltpu.CompilerParams(use_tc_tiling_on_sc=False,
                                  needs_layout_passes=False)

_hop = pl.kernel(
    _hop_body,
    out_type=jax.ShapeDtypeStruct((NC, N_NODES, D), jnp.float32),
    mesh=_MESH,
    compiler_params=_SC_PARAMS,
    scratch_types=[
        pltpu.VMEM((BLK, C), jnp.int32),
        pltpu.VMEM((BLK, C), jnp.int32),
        pltpu.VMEM((BLK, C), jnp.float32),
        pltpu.VMEM((BLK, C), jnp.float32),
        pltpu.VMEM((C,), jnp.int32),
        pltpu.VMEM((C,), jnp.int32),
        pltpu.VMEM((C,), jnp.int32),
        pltpu.VMEM((C,), jnp.int32),
        pltpu.VMEM((C,), jnp.int32),
        pltpu.VMEM((C,), jnp.int32),
        pltpu.VMEM((C,), jnp.int32),
        pltpu.VMEM((C,), jnp.int32),
        pltpu.VMEM((C,), jnp.float32),
        pltpu.VMEM((C,), jnp.float32),
        pltpu.VMEM((C,), jnp.float32),
        pltpu.VMEM((C,), jnp.float32),
        pltpu.VMEM((C, DW), jnp.int32),
        pltpu.VMEM((C, DW), jnp.int32),
        pltpu.VMEM((C, D), jnp.float32),
        pltpu.VMEM((C, D), jnp.float32),
        pltpu.VMEM((ZB, D), jnp.float32),
        pltpu.VMEM_SHARED((N_NODES, D), jnp.float32),
        pltpu.SemaphoreType.DMA,
        pltpu.SemaphoreType.DMA,
        pltpu.SemaphoreType.DMA,
        pltpu.SemaphoreType.DMA,
        pltpu.SemaphoreType.DMA,
        pltpu.SemaphoreType.DMA,
    ],
)

_combine = pl.kernel(
    _combine_body,
    out_type=jax.ShapeDtypeStruct((N_NODES, D), jnp.float32),
    mesh=_MESH,
    compiler_params=_SC_PARAMS,
    scratch_types=[
        pltpu.VMEM((ZR, D), jnp.float32),
        pltpu.VMEM((ZR, D), jnp.float32),
    ],
)


def _pad_chunks(x):
    x = x.reshape(NW, EPT)
    x = jnp.pad(x, ((0, 0), (0, EPAD - EPT)))
    return x.reshape(NW, NBLK, BLK, C)


def _pack16(x):
    return lax.bitcast_convert_type(
        x.astype(jnp.bfloat16).reshape(N_NODES, DW, 2),
        jnp.int32).reshape(N_NODES, DW)


def kernel(embed, edge_index, trend):
    row = edge_index[0].astype(jnp.int32)
    col = edge_index[1].astype(jnp.int32)
    pk = _pad_chunks(row | (col << 16))
    trf = _pad_chunks(trend.astype(jnp.float32))

    aggs = []
    agg = embed
    for _ in range(N_HOPS_K):
        part = _hop(_pack16(agg), pk, trf)
        agg = _combine(part)
        aggs.append(agg)

    a1 = aggs[0][:, _INV1]
    a2 = aggs[1][:, _INV2]
    a3 = aggs[2][:, _INV3]
    return jnp.stack([embed, a1, a2, a3], axis=1)


# speedup vs baseline: 1.1215x; 1.1215x over previous
import jax
import jax.numpy as jnp
from jax import lax
from jax.experimental import pallas as pl
from jax.experimental.pallas import tpu as pltpu
from jax.experimental.pallas import tpu_sc as plsc

N_NODES = 10000
D = 128
E = 320000
N_HOPS_K = 3

NC = 2
NS = 16
NW = NC * NS
EPT = E // NW
CW = 128
NROW = 80
C = 64
NCHUNK = 160
EPAD = NROW * CW
RPT = 624
ZB = 16
NZ = RPT // ZB
ZR = 104
TAIL = N_NODES - NS * RPT
TAIL_OFF = NS * RPT

RPC = 312
CTAIL_OFF = NW * RPC

_MESH = plsc.VectorSubcoreMesh(
    core_axis_name="c", subcore_axis_name="s", num_cores=NC, num_subcores=NS
)


def _hop_body(agg, pkr, trendr, part,
              pk_v, trend_v, rb0, rb1, cb0, cb1, gb0, gb1, zbuf, acc,
              gs0, gs1):
    cid = lax.axis_index("c")
    sid = lax.axis_index("s")
    wid = cid * NS + sid

    rb = (rb0, rb1)
    cb = (cb0, cb1)
    gb = (gb0, gb1)
    gsem = (gs0, gs1)

    def zb(j, carry):
        for k in range(D // 16):
            zbuf[j, pl.ds(k * 16, 16)] = jnp.zeros((16,), jnp.float32)
        return carry
    lax.fori_loop(0, ZB, zb, 0)

    pltpu.sync_copy(pkr.at[wid], pk_v)
    pltpu.sync_copy(trendr.at[wid], trend_v)

    def za(k, carry):
        pltpu.sync_copy(zbuf, acc.at[pl.ds(sid * RPT + k * ZB, ZB)])
        return carry
    lax.fori_loop(0, NZ, za, 0)

    @pl.when(sid == NS - 1)
    def _():
        pltpu.sync_copy(zbuf.at[pl.ds(0, TAIL)], acc.at[pl.ds(TAIL_OFF, TAIL)])
    plsc.subcore_barrier()

    def unpack(i, h, p):
        for w in range(C // 16):
            v = pk_v[i, pl.ds(64 * h + w * 16, 16)]
            rb[p][pl.ds(w * 16, 16)] = v & 0xFFFF
            cb[p][pl.ds(w * 16, 16)] = lax.shift_right_logical(v, 16)

    def gstart(p):
        pltpu.async_copy(agg.at[rb[p]], gb[p], gsem[p])

    def gwait(p):
        pltpu.make_async_copy(agg.at[rb[p]], gb[p], gsem[p]).wait()

    def work(i, h, p):
        b = gb[p]

        def grp(j16, carry):
            t16 = trend_v[i, pl.ds(64 * h + j16 * 16, 16)]
            for jj in range(16):
                j = j16 * 16 + jj
                tb = lax.broadcast(t16[jj], (16,))
                for k in range(D // 16):
                    b[j, pl.ds(k * 16, 16)] = b[j, pl.ds(k * 16, 16)] * tb
            return carry
        lax.fori_loop(0, C // 16, grp, 0)
        pltpu.sync_copy(b, acc.at[cb[p]], add=True)

    unpack(0, 0, 0)
    gstart(0)

    def pair(i2, carry):
        i = i2
        unpack(i, 1, 1)
        gstart(1)
        gwait(0)
        work(i, 0, 0)

        @pl.when(i + 1 < NROW)
        def _():
            unpack(i + 1, 0, 0)
            gstart(0)
        gwait(1)
        work(i, 1, 1)
        return carry
    lax.fori_loop(0, NROW, pair, 0)

    plsc.subcore_barrier()
    pltpu.sync_copy(acc.at[pl.ds(sid * RPT, RPT)],
                    part.at[cid, pl.ds(sid * RPT, RPT)])

    @pl.when(sid == NS - 1)
    def _():
        pltpu.sync_copy(acc.at[pl.ds(TAIL_OFF, TAIL)],
                        part.at[cid, pl.ds(TAIL_OFF, TAIL)])


def _combine_body(part, out, b0, b1):
    cid = lax.axis_index("c")
    sid = lax.axis_index("s")
    wid = cid * NS + sid

    def _sum_rows(nrows, off):
        pltpu.sync_copy(part.at[0, pl.ds(off, nrows)], b0.at[pl.ds(0, nrows)])
        pltpu.sync_copy(part.at[1, pl.ds(off, nrows)], b1.at[pl.ds(0, nrows)])

        def addrow(j, c2):
            for kk in range(D // 16):
                b0[j, pl.ds(kk * 16, 16)] = (
                    b0[j, pl.ds(kk * 16, 16)] + b1[j, pl.ds(kk * 16, 16)])
            return c2
        lax.fori_loop(0, nrows, addrow, 0)
        pltpu.sync_copy(b0.at[pl.ds(0, nrows)], out.at[pl.ds(off, nrows)])

    def ck(k, carry):
        _sum_rows(ZR, wid * RPC + k * ZR)
        return carry
    lax.fori_loop(0, RPC // ZR, ck, 0)

    @pl.when(wid == NW - 1)
    def _():
        _sum_rows(TAIL, CTAIL_OFF)


_hop = pl.kernel(
    _hop_body,
    out_type=jax.ShapeDtypeStruct((NC, N_NODES, D), jnp.float32),
    mesh=_MESH,
    scratch_types=[
        pltpu.VMEM((NROW, CW), jnp.int32),
        pltpu.VMEM((NROW, CW), jnp.float32),
        pltpu.VMEM((C,), jnp.int32),
        pltpu.VMEM((C,), jnp.int32),
        pltpu.VMEM((C,), jnp.int32),
        pltpu.VMEM((C,), jnp.int32),
        pltpu.VMEM((C, D), jnp.float32),
        pltpu.VMEM((C, D), jnp.float32),
        pltpu.VMEM((ZB, D), jnp.float32),
        pltpu.VMEM_SHARED((N_NODES, D), jnp.float32),
        pltpu.SemaphoreType.DMA,
        pltpu.SemaphoreType.DMA,
    ],
)

_combine = pl.kernel(
    _combine_body,
    out_type=jax.ShapeDtypeStruct((N_NODES, D), jnp.float32),
    mesh=_MESH,
    scratch_types=[
        pltpu.VMEM((ZR, D), jnp.float32),
        pltpu.VMEM((ZR, D), jnp.float32),
    ],
)


def _pad_chunks(x):
    x = x.reshape(NW, EPT)
    x = jnp.pad(x, ((0, 0), (0, EPAD - EPT)))
    return x.reshape(NW, NROW, CW)


def kernel(embed, edge_index, trend):
    row = edge_index[0].astype(jnp.int32)
    col = edge_index[1].astype(jnp.int32)
    pk = _pad_chunks(row | (col << 16))
    trf = _pad_chunks(trend.astype(jnp.float32))

    embs = [embed]
    agg = embed
    for _ in range(N_HOPS_K):
        part = _hop(agg, pk, trf)
        agg = _combine(part)
        embs.append(agg)
    return jnp.stack(embs, axis=1)


# speedup vs baseline: 1.3625x; 1.2149x over previous
import jax
import jax.numpy as jnp
from jax import lax
from jax.experimental import pallas as pl
from jax.experimental.pallas import tpu as pltpu
from jax.experimental.pallas import tpu_sc as plsc

N_NODES = 10000
D = 128
E = 320000
N_HOPS_K = 3

NC = 2
NS = 16
NW = NC * NS
EPT = E // NW
C = 128
NCHUNK = 79
EPAD = NCHUNK * C
RPT = 624
ZB = 16
NZ = RPT // ZB
ZR = 104
TAIL = N_NODES - NS * RPT
TAIL_OFF = NS * RPT

RPC = 312
CTAIL_OFF = NW * RPC

_MESH = plsc.VectorSubcoreMesh(
    core_axis_name="c", subcore_axis_name="s", num_cores=NC, num_subcores=NS
)


def _hop_body(agg, rowr, colr, trendr, part,
              row_v, col_v, trend_v, buf, zbuf, acc, sem):
    cid = lax.axis_index("c")
    sid = lax.axis_index("s")
    wid = cid * NS + sid

    def zb(j, carry):
        for k in range(D // 16):
            zbuf[j, pl.ds(k * 16, 16)] = jnp.zeros((16,), jnp.float32)
        return carry
    lax.fori_loop(0, ZB, zb, 0)

    pltpu.sync_copy(rowr.at[wid], row_v)
    pltpu.sync_copy(colr.at[wid], col_v)
    pltpu.sync_copy(trendr.at[wid], trend_v)

    def za(k, carry):
        pltpu.sync_copy(zbuf, acc.at[pl.ds(sid * RPT + k * ZB, ZB)])
        return carry
    lax.fori_loop(0, NZ, za, 0)

    @pl.when(sid == NS - 1)
    def _():
        pltpu.sync_copy(zbuf.at[pl.ds(0, TAIL)], acc.at[pl.ds(TAIL_OFF, TAIL)])
    plsc.subcore_barrier()

    def chunk(i, carry):
        pltpu.async_copy(agg.at[row_v.at[i]], buf, sem).wait()
        for j16 in range(C // 16):
            t16 = trend_v[i, pl.ds(j16 * 16, 16)]
            for jj in range(16):
                j = j16 * 16 + jj
                tb = lax.broadcast(t16[jj], (16,))
                for k in range(D // 16):
                    buf[j, pl.ds(k * 16, 16)] = buf[j, pl.ds(k * 16, 16)] * tb
        pltpu.sync_copy(buf, acc.at[col_v.at[i]], add=True)
        return carry
    lax.fori_loop(0, NCHUNK, chunk, 0)

    plsc.subcore_barrier()
    pltpu.sync_copy(acc.at[pl.ds(sid * RPT, RPT)],
                    part.at[cid, pl.ds(sid * RPT, RPT)])

    @pl.when(sid == NS - 1)
    def _():
        pltpu.sync_copy(acc.at[pl.ds(TAIL_OFF, TAIL)],
                        part.at[cid, pl.ds(TAIL_OFF, TAIL)])


def _combine_body(part, out, b0, b1):
    cid = lax.axis_index("c")
    sid = lax.axis_index("s")
    wid = cid * NS + sid

    def _sum_rows(nrows, off):
        pltpu.sync_copy(part.at[0, pl.ds(off, nrows)], b0.at[pl.ds(0, nrows)])
        pltpu.sync_copy(part.at[1, pl.ds(off, nrows)], b1.at[pl.ds(0, nrows)])

        def addrow(j, c2):
            for kk in range(D // 16):
                b0[j, pl.ds(kk * 16, 16)] = (
                    b0[j, pl.ds(kk * 16, 16)] + b1[j, pl.ds(kk * 16, 16)])
            return c2
        lax.fori_loop(0, nrows, addrow, 0)
        pltpu.sync_copy(b0.at[pl.ds(0, nrows)], out.at[pl.ds(off, nrows)])

    def ck(k, carry):
        _sum_rows(ZR, wid * RPC + k * ZR)
        return carry
    lax.fori_loop(0, RPC // ZR, ck, 0)

    @pl.when(wid == NW - 1)
    def _():
        _sum_rows(TAIL, CTAIL_OFF)


_hop = pl.kernel(
    _hop_body,
    out_type=jax.ShapeDtypeStruct((NC, N_NODES, D), jnp.float32),
    mesh=_MESH,
    scratch_types=[
        pltpu.VMEM((NCHUNK, C), jnp.int32),
        pltpu.VMEM((NCHUNK, C), jnp.int32),
        pltpu.VMEM((NCHUNK, C), jnp.float32),
        pltpu.VMEM((C, D), jnp.float32),
        pltpu.VMEM((ZB, D), jnp.float32),
        pltpu.VMEM_SHARED((N_NODES, D), jnp.float32),
        pltpu.SemaphoreType.DMA,
    ],
)

_combine = pl.kernel(
    _combine_body,
    out_type=jax.ShapeDtypeStruct((N_NODES, D), jnp.float32),
    mesh=_MESH,
    scratch_types=[
        pltpu.VMEM((ZR, D), jnp.float32),
        pltpu.VMEM((ZR, D), jnp.float32),
    ],
)


def _pad_chunks(x):
    x = x.reshape(NW, EPT)
    x = jnp.pad(x, ((0, 0), (0, EPAD - EPT)))
    return x.reshape(NW, NCHUNK, C)


def kernel(embed, edge_index, trend):
    row = _pad_chunks(edge_index[0].astype(jnp.int32))
    col = _pad_chunks(edge_index[1].astype(jnp.int32))
    tr = _pad_chunks(trend.astype(jnp.float32))

    embs = [embed]
    agg = embed
    for _ in range(N_HOPS_K):
        part = _hop(agg, row, col, tr)
        agg = _combine(part)
        embs.append(agg)
    return jnp.stack(embs, axis=1)


# speedup vs baseline: 1.3741x; 1.0085x over previous
import jax
import jax.numpy as jnp
from jax import lax
from jax.experimental import pallas as pl
from jax.experimental.pallas import tpu as pltpu
from jax.experimental.pallas import tpu_sc as plsc

N_NODES = 10000
D = 128
E = 320000
N_HOPS_K = 3

NC = 2
NS = 16
NW = NC * NS
EPT = E // NW
C = 128
NCHUNK = 79
EPAD = NCHUNK * C
RPT = 624
ZB = 16
NZ = RPT // ZB
ZR = 104
TAIL = N_NODES - NS * RPT
TAIL_OFF = NS * RPT

RPC = 312
CTAIL_OFF = NW * RPC

_MESH = plsc.VectorSubcoreMesh(
    core_axis_name="c", subcore_axis_name="s", num_cores=NC, num_subcores=NS
)


def _hop_body(agg, rowr, colr, trendr, part,
              row_v, col_v, trend_v, buf, zbuf, acc, sem):
    cid = lax.axis_index("c")
    sid = lax.axis_index("s")
    wid = cid * NS + sid

    pltpu.async_copy(rowr.at[wid], row_v, sem)
    pltpu.async_copy(colr.at[wid], col_v, sem)
    pltpu.async_copy(trendr.at[wid], trend_v, sem)

    def zb(j, carry):
        for k in range(D // 16):
            zbuf[j, pl.ds(k * 16, 16)] = jnp.zeros((16,), jnp.float32)
        return carry
    lax.fori_loop(0, ZB, zb, 0)

    def za(k, carry):
        pltpu.sync_copy(zbuf, acc.at[pl.ds(sid * RPT + k * ZB, ZB)])
        return carry
    lax.fori_loop(0, NZ, za, 0)

    pltpu.make_async_copy(rowr.at[wid], row_v, sem).wait()
    pltpu.make_async_copy(colr.at[wid], col_v, sem).wait()
    pltpu.make_async_copy(trendr.at[wid], trend_v, sem).wait()

    @pl.when(sid == NS - 1)
    def _():
        pltpu.sync_copy(zbuf.at[pl.ds(0, TAIL)], acc.at[pl.ds(TAIL_OFF, TAIL)])
    plsc.subcore_barrier()

    def chunk(i, carry):
        pltpu.async_copy(agg.at[row_v.at[i]], buf, sem).wait()
        for j16 in range(C // 16):
            t16 = trend_v[i, pl.ds(j16 * 16, 16)]
            for jj in range(16):
                j = j16 * 16 + jj
                tb = lax.broadcast(t16[jj], (16,))
                for k in range(D // 16):
                    buf[j, pl.ds(k * 16, 16)] = buf[j, pl.ds(k * 16, 16)] * tb
        pltpu.sync_copy(buf, acc.at[col_v.at[i]], add=True)
        return carry
    lax.fori_loop(0, NCHUNK, chunk, 0)

    plsc.subcore_barrier()
    pltpu.sync_copy(acc.at[pl.ds(sid * RPT, RPT)],
                    part.at[cid, pl.ds(sid * RPT, RPT)])

    @pl.when(sid == NS - 1)
    def _():
        pltpu.sync_copy(acc.at[pl.ds(TAIL_OFF, TAIL)],
                        part.at[cid, pl.ds(TAIL_OFF, TAIL)])


def _combine_body(part, out, b0, b1):
    cid = lax.axis_index("c")
    sid = lax.axis_index("s")
    wid = cid * NS + sid

    def _sum_rows(nrows, off):
        pltpu.sync_copy(part.at[0, pl.ds(off, nrows)], b0.at[pl.ds(0, nrows)])
        pltpu.sync_copy(part.at[1, pl.ds(off, nrows)], b1.at[pl.ds(0, nrows)])

        def addrow(j, c2):
            for kk in range(D // 16):
                b0[j, pl.ds(kk * 16, 16)] = (
                    b0[j, pl.ds(kk * 16, 16)] + b1[j, pl.ds(kk * 16, 16)])
            return c2
        lax.fori_loop(0, nrows, addrow, 0)
        pltpu.sync_copy(b0.at[pl.ds(0, nrows)], out.at[pl.ds(off, nrows)])

    def ck(k, carry):
        _sum_rows(ZR, wid * RPC + k * ZR)
        return carry
    lax.fori_loop(0, RPC // ZR, ck, 0)

    @pl.when(wid == NW - 1)
    def _():
        _sum_rows(TAIL, CTAIL_OFF)


_hop = pl.kernel(
    _hop_body,
    out_type=jax.ShapeDtypeStruct((NC, N_NODES, D), jnp.float32),
    mesh=_MESH,
    scratch_types=[
        pltpu.VMEM((NCHUNK, C), jnp.int32),
        pltpu.VMEM((NCHUNK, C), jnp.int32),
        pltpu.VMEM((NCHUNK, C), jnp.float32),
        pltpu.VMEM((C, D), jnp.float32),
        pltpu.VMEM((ZB, D), jnp.float32),
        pltpu.VMEM_SHARED((N_NODES, D), jnp.float32),
        pltpu.SemaphoreType.DMA,
    ],
)

_combine = pl.kernel(
    _combine_body,
    out_type=jax.ShapeDtypeStruct((N_NODES, D), jnp.float32),
    mesh=_MESH,
    scratch_types=[
        pltpu.VMEM((ZR, D), jnp.float32),
        pltpu.VMEM((ZR, D), jnp.float32),
    ],
)


def _pad_chunks(x):
    x = x.reshape(NW, EPT)
    x = jnp.pad(x, ((0, 0), (0, EPAD - EPT)))
    return x.reshape(NW, NCHUNK, C)


def kernel(embed, edge_index, trend):
    row = _pad_chunks(edge_index[0].astype(jnp.int32))
    col = _pad_chunks(edge_index[1].astype(jnp.int32))
    tr = _pad_chunks(trend.astype(jnp.float32))

    embs = [embed]
    agg = embed
    for _ in range(N_HOPS_K):
        part = _hop(agg, row, col, tr)
        agg = _combine(part)
        embs.append(agg)
    return jnp.stack(embs, axis=1)
